# Initial kernel scaffold; baseline (speedup 1.0000x reference)
import jax
import jax.numpy as jnp
from jax.experimental import pallas as pl


def kernel(x, edge_index, Wl1, bl1, Wr1, Wl2, bl2, Wr2, Wfc, bfc):
    raise NotImplementedError("write your pallas kernel here")


# speedup vs baseline: 5.3340x; 5.3340x over previous
import functools

import jax
import jax.numpy as jnp
from jax import lax
from jax.experimental import pallas as pl
from jax.experimental.pallas import tpu as pltpu
from jax.experimental.pallas import tpu_sc as plsc

N = 10000
E = 320000
IN_CH = 128
HID_CH = 256
OUT_CH = 64

NC = 2
NS = 16
NW = NC * NS

K1 = 125
C1 = E // NW // K1
K2 = 125
C2 = E // NS // K2
NP = 10240
ROWS_T = NP // NS

@functools.lru_cache(maxsize=None)
def _sc_kernels():
    mesh = plsc.VectorSubcoreMesh(core_axis_name="c", subcore_axis_name="s",
                                  num_cores=NC, num_subcores=NS)

    @functools.partial(
        pl.kernel,
        out_type=(
            jax.ShapeDtypeStruct((NC, NP, IN_CH), jnp.float32),
            jax.ShapeDtypeStruct((NC, NP), jnp.float32),
        ),
        mesh=mesh,
        scratch_types=[
            pltpu.VMEM((K1,), jnp.int32),
            pltpu.VMEM((K1,), jnp.int32),
            pltpu.VMEM((K1, IN_CH), jnp.float32),
            pltpu.VMEM((128,), jnp.float32),
            pltpu.VMEM_SHARED((NP, IN_CH), jnp.float32),
            pltpu.VMEM_SHARED((NP,), jnp.float32),
        ],
    )
    def sc_layer1(x_hbm, src3, dst3, zf, z1, aggp, cntp,
                  srcv, dstv, rowsv, onesv, acc, cacc):
        c = lax.axis_index("c")
        s = lax.axis_index("s")
        w = c * NS + s
        r0 = s * ROWS_T
        pltpu.sync_copy(zf.at[pl.ds(r0, ROWS_T)], acc.at[pl.ds(r0, ROWS_T)])
        pltpu.sync_copy(z1.at[pl.ds(r0, ROWS_T)], cacc.at[pl.ds(r0, ROWS_T)])
        ones16 = jnp.ones((16,), jnp.float32)
        for i in range(8):
            onesv[pl.ds(i * 16, 16)] = ones16
        plsc.subcore_barrier()

        def body(j, carry):
            pltpu.sync_copy(src3.at[w].at[j], srcv)
            pltpu.sync_copy(dst3.at[w].at[j], dstv)
            pltpu.sync_copy(x_hbm.at[srcv], rowsv)
            pltpu.sync_copy(rowsv, acc.at[dstv], add=True)
            pltpu.sync_copy(onesv.at[pl.ds(0, K1)], cacc.at[dstv],
                            add=True)
            return carry

        lax.fori_loop(0, C1, body, 0)
        plsc.subcore_barrier()
        pltpu.sync_copy(acc.at[pl.ds(r0, ROWS_T)],
                        aggp.at[c].at[pl.ds(r0, ROWS_T)])
        pltpu.sync_copy(cacc.at[pl.ds(r0, ROWS_T)],
                        cntp.at[c].at[pl.ds(r0, ROWS_T)])

    @functools.partial(
        pl.kernel,
        out_type=jax.ShapeDtypeStruct((NC, NP, IN_CH), jnp.float32),
        mesh=mesh,
        scratch_types=[
            pltpu.VMEM((K2,), jnp.int32),
            pltpu.VMEM((K2,), jnp.int32),
            pltpu.VMEM((K2, IN_CH), jnp.float32),
            pltpu.VMEM_SHARED((NP, IN_CH), jnp.float32),
        ],
    )
    def sc_layer2(h1s_hbm, srcB, dstB, zf, agg2,
                  srcv, dstv, rowsv, acc):
        c = lax.axis_index("c")
        s = lax.axis_index("s")
        r0 = s * ROWS_T
        pltpu.sync_copy(zf.at[pl.ds(r0, ROWS_T)], acc.at[pl.ds(r0, ROWS_T)])
        plsc.subcore_barrier()

        def body(j, carry):
            pltpu.sync_copy(srcB.at[s].at[j], srcv)
            pltpu.sync_copy(dstB.at[s].at[j], dstv)
            pltpu.sync_copy(h1s_hbm.at[c].at[srcv], rowsv)
            pltpu.sync_copy(rowsv, acc.at[dstv], add=True)
            return carry

        lax.fori_loop(0, C2, body, 0)
        plsc.subcore_barrier()
        pltpu.sync_copy(acc.at[pl.ds(r0, ROWS_T)],
                        agg2.at[c].at[pl.ds(r0, ROWS_T)])

    return sc_layer1, sc_layer2


BM = 2000


def _d1_body(aggp, rcnt, x, wl, bl, wr, h1s):
    agg = aggp[0] + aggp[1]
    mw = lax.dot(agg, wl[...], precision=lax.Precision.HIGHEST) * rcnt[...]
    h = mw + bl[...] + lax.dot(x[...], wr[...], precision=lax.Precision.HIGHEST)
    h = jnp.maximum(h, 0.0)
    h1s[0] = h[:, :IN_CH]
    h1s[1] = h[:, IN_CH:]


def _d2_body(agg2, rcnt, h1s, wl, bl, wr, wfc, bfc, out):
    agg = jnp.concatenate([agg2[0], agg2[1]], axis=1)
    h1 = jnp.concatenate([h1s[0], h1s[1]], axis=1)
    mw = lax.dot(agg, wl[...], precision=lax.Precision.HIGHEST) * rcnt[...]
    h2 = mw + bl[...] + lax.dot(h1, wr[...], precision=lax.Precision.HIGHEST)
    h2 = jnp.maximum(h2, 0.0)
    z = lax.dot(h2, wfc[...], precision=lax.Precision.HIGHEST) + bfc[...]
    m = jnp.max(z, axis=1, keepdims=True)
    e = z - m
    out[...] = e - jnp.log(jnp.sum(jnp.exp(e), axis=1, keepdims=True))


_GM = N // BM

_d1_call = pl.pallas_call(
    _d1_body,
    grid=(_GM,),
    in_specs=[
        pl.BlockSpec((2, BM, IN_CH), lambda i: (0, i, 0)),
        pl.BlockSpec((BM, 1), lambda i: (i, 0)),
        pl.BlockSpec((BM, IN_CH), lambda i: (i, 0)),
        pl.BlockSpec((IN_CH, HID_CH), lambda i: (0, 0)),
        pl.BlockSpec((1, HID_CH), lambda i: (0, 0)),
        pl.BlockSpec((IN_CH, HID_CH), lambda i: (0, 0)),
    ],
    out_specs=pl.BlockSpec((2, BM, IN_CH), lambda i: (0, i, 0)),
    out_shape=jax.ShapeDtypeStruct((2, N, IN_CH), jnp.float32),
)

_d2_call = pl.pallas_call(
    _d2_body,
    grid=(_GM,),
    in_specs=[
        pl.BlockSpec((2, BM, IN_CH), lambda i: (0, i, 0)),
        pl.BlockSpec((BM, 1), lambda i: (i, 0)),
        pl.BlockSpec((2, BM, IN_CH), lambda i: (0, i, 0)),
        pl.BlockSpec((HID_CH, HID_CH), lambda i: (0, 0)),
        pl.BlockSpec((1, HID_CH), lambda i: (0, 0)),
        pl.BlockSpec((HID_CH, HID_CH), lambda i: (0, 0)),
        pl.BlockSpec((HID_CH, OUT_CH), lambda i: (0, 0)),
        pl.BlockSpec((1, OUT_CH), lambda i: (0, 0)),
    ],
    out_specs=pl.BlockSpec((BM, OUT_CH), lambda i: (i, 0)),
    out_shape=jax.ShapeDtypeStruct((N, OUT_CH), jnp.float32),
)


def kernel(x, edge_index, Wl1, bl1, Wr1, Wl2, bl2, Wr2, Wfc, bfc):
    src = edge_index[0].astype(jnp.int32)
    dst = edge_index[1].astype(jnp.int32)
    src3 = src.reshape(NW, C1, K1)
    dst3 = dst.reshape(NW, C1, K1)
    srcB = src.reshape(NS, C2, K2)
    dstB = dst.reshape(NS, C2, K2)
    zf = jnp.zeros((NP, IN_CH), jnp.float32)
    z1 = jnp.zeros((NP,), jnp.float32)

    sc_layer1, sc_layer2 = _sc_kernels()
    aggp, cntp = sc_layer1(x, src3, dst3, zf, z1)
    rcnt = (1.0 / jnp.clip(cntp[0] + cntp[1], 1.0, None))[:, None]
    h1s = _d1_call(aggp, rcnt, x, Wl1, bl1.reshape(1, -1), Wr1)
    agg2 = sc_layer2(h1s, srcB, dstB, zf)
    out = _d2_call(agg2, rcnt, h1s, Wl2, bl2.reshape(1, -1), Wr2,
                   Wfc, bfc.reshape(1, -1))
    return out


# speedup vs baseline: 8.0688x; 1.5127x over previous
import functools

import jax
import jax.numpy as jnp
from jax import lax
from jax.experimental import pallas as pl
from jax.experimental.pallas import tpu as pltpu
from jax.experimental.pallas import tpu_sc as plsc

N = 10000
E = 320000
IN_CH = 128
HID_CH = 256
OUT_CH = 64

NC = 2
NS = 16
NW = NC * NS

K1 = 125
C1 = E // NW // K1
K2 = 125
C2 = E // NS // K2
NP = 10240
ROWS_T = NP // NS

@functools.lru_cache(maxsize=None)
def _sc_kernels():
    mesh = plsc.VectorSubcoreMesh(core_axis_name="c", subcore_axis_name="s",
                                  num_cores=NC, num_subcores=NS)

    @functools.partial(
        pl.kernel,
        out_type=(
            jax.ShapeDtypeStruct((NC, NP, IN_CH), jnp.float32),
            jax.ShapeDtypeStruct((NC, NP), jnp.float32),
        ),
        mesh=mesh,
        scratch_types=[
            pltpu.VMEM((K1,), jnp.int32),
            pltpu.VMEM((K1,), jnp.int32),
            pltpu.VMEM((K1,), jnp.int32),
            pltpu.VMEM((K1,), jnp.int32),
            pltpu.VMEM((K1, IN_CH), jnp.float32),
            pltpu.VMEM((K1, IN_CH), jnp.float32),
            pltpu.VMEM((128,), jnp.float32),
            pltpu.VMEM_SHARED((NP, IN_CH), jnp.float32),
            pltpu.VMEM_SHARED((NP,), jnp.float32),
            pltpu.SemaphoreType.DMA,
            pltpu.SemaphoreType.DMA,
            pltpu.SemaphoreType.DMA,
            pltpu.SemaphoreType.DMA,
        ],
    )
    def sc_layer1(x_hbm, src3, dst3, zf, z1, aggp, cntp,
                  si0, si1, di0, di1, rv0, rv1, onesv, acc, cacc,
                  gs0, gs1, ss0, ss1):
        c = lax.axis_index("c")
        s = lax.axis_index("s")
        w = c * NS + s
        r0 = s * ROWS_T
        si = (si0, si1)
        di = (di0, di1)
        rv = (rv0, rv1)
        gs = (gs0, gs1)
        ss = (ss0, ss1)
        pltpu.sync_copy(zf.at[pl.ds(r0, ROWS_T)], acc.at[pl.ds(r0, ROWS_T)])
        pltpu.sync_copy(z1.at[pl.ds(r0, ROWS_T)], cacc.at[pl.ds(r0, ROWS_T)])
        ones16 = jnp.ones((16,), jnp.float32)
        for i in range(8):
            onesv[pl.ds(i * 16, 16)] = ones16
        plsc.subcore_barrier()

        def issue_gather(j, b):
            pltpu.sync_copy(src3.at[w].at[j], si[b])
            pltpu.sync_copy(dst3.at[w].at[j], di[b])
            pltpu.async_copy(x_hbm.at[si[b]], rv[b], gs[b])

        issue_gather(0, 0)
        issue_gather(1, 1)

        def body(k, carry):
            for b in range(2):
                j = 2 * k + b
                pltpu.make_async_copy(x_hbm.at[si[b]], rv[b], gs[b]).wait()
                pltpu.async_copy(rv[b], acc.at[di[b]], ss[b], add=True)
                pltpu.sync_copy(onesv.at[pl.ds(0, K1)], cacc.at[di[b]],
                                add=True)
            for b in range(2):
                pltpu.make_async_copy(rv[b], acc.at[di[b]], ss[b]).wait()
                issue_gather(2 * k + 2 + b, b)
            return carry

        lax.fori_loop(0, C1 // 2 - 1, body, 0)
        for b in range(2):
            pltpu.make_async_copy(x_hbm.at[si[b]], rv[b], gs[b]).wait()
            pltpu.async_copy(rv[b], acc.at[di[b]], ss[b], add=True)
            pltpu.sync_copy(onesv.at[pl.ds(0, K1)], cacc.at[di[b]], add=True)
        for b in range(2):
            pltpu.make_async_copy(rv[b], acc.at[di[b]], ss[b]).wait()
        plsc.subcore_barrier()
        pltpu.sync_copy(acc.at[pl.ds(r0, ROWS_T)],
                        aggp.at[c].at[pl.ds(r0, ROWS_T)])
        pltpu.sync_copy(cacc.at[pl.ds(r0, ROWS_T)],
                        cntp.at[c].at[pl.ds(r0, ROWS_T)])

    @functools.partial(
        pl.kernel,
        out_type=jax.ShapeDtypeStruct((NC, NP, IN_CH), jnp.float32),
        mesh=mesh,
        scratch_types=[
            pltpu.VMEM((K2,), jnp.int32),
            pltpu.VMEM((K2,), jnp.int32),
            pltpu.VMEM((K2,), jnp.int32),
            pltpu.VMEM((K2,), jnp.int32),
            pltpu.VMEM((K2, IN_CH), jnp.float32),
            pltpu.VMEM((K2, IN_CH), jnp.float32),
            pltpu.VMEM_SHARED((NP, IN_CH), jnp.float32),
            pltpu.SemaphoreType.DMA,
            pltpu.SemaphoreType.DMA,
            pltpu.SemaphoreType.DMA,
            pltpu.SemaphoreType.DMA,
        ],
    )
    def sc_layer2(h1s_hbm, srcB, dstB, zf, agg2,
                  si0, si1, di0, di1, rv0, rv1, acc, gs0, gs1, ss0, ss1):
        c = lax.axis_index("c")
        s = lax.axis_index("s")
        r0 = s * ROWS_T
        si = (si0, si1)
        di = (di0, di1)
        rv = (rv0, rv1)
        gs = (gs0, gs1)
        ss = (ss0, ss1)
        pltpu.sync_copy(zf.at[pl.ds(r0, ROWS_T)], acc.at[pl.ds(r0, ROWS_T)])
        plsc.subcore_barrier()

        def issue_gather(j, b):
            pltpu.sync_copy(srcB.at[s].at[j], si[b])
            pltpu.sync_copy(dstB.at[s].at[j], di[b])
            pltpu.async_copy(h1s_hbm.at[c].at[si[b]], rv[b], gs[b])

        issue_gather(0, 0)
        issue_gather(1, 1)

        def body(k, carry):
            for b in range(2):
                pltpu.make_async_copy(h1s_hbm.at[c].at[si[b]], rv[b],
                                      gs[b]).wait()
                pltpu.async_copy(rv[b], acc.at[di[b]], ss[b], add=True)
            for b in range(2):
                pltpu.make_async_copy(rv[b], acc.at[di[b]], ss[b]).wait()
                issue_gather(2 * k + 2 + b, b)
            return carry

        lax.fori_loop(0, C2 // 2 - 1, body, 0)
        for b in range(2):
            pltpu.make_async_copy(h1s_hbm.at[c].at[si[b]], rv[b],
                                  gs[b]).wait()
            pltpu.async_copy(rv[b], acc.at[di[b]], ss[b], add=True)
        for b in range(2):
            pltpu.make_async_copy(rv[b], acc.at[di[b]], ss[b]).wait()
        plsc.subcore_barrier()
        pltpu.sync_copy(acc.at[pl.ds(r0, ROWS_T)],
                        agg2.at[c].at[pl.ds(r0, ROWS_T)])

    return sc_layer1, sc_layer2


BM = 2000


def _d1_body(aggp, rcnt, x, wl, bl, wr, h1s):
    agg = aggp[0] + aggp[1]
    mw = lax.dot(agg, wl[...], precision=lax.Precision.HIGHEST) * rcnt[...]
    h = mw + bl[...] + lax.dot(x[...], wr[...], precision=lax.Precision.HIGHEST)
    h = jnp.maximum(h, 0.0)
    h1s[0] = h[:, :IN_CH]
    h1s[1] = h[:, IN_CH:]


def _d2_body(agg2, rcnt, h1s, wl, bl, wr, wfc, bfc, out):
    agg = jnp.concatenate([agg2[0], agg2[1]], axis=1)
    h1 = jnp.concatenate([h1s[0], h1s[1]], axis=1)
    mw = lax.dot(agg, wl[...], precision=lax.Precision.HIGHEST) * rcnt[...]
    h2 = mw + bl[...] + lax.dot(h1, wr[...], precision=lax.Precision.HIGHEST)
    h2 = jnp.maximum(h2, 0.0)
    z = lax.dot(h2, wfc[...], precision=lax.Precision.HIGHEST) + bfc[...]
    m = jnp.max(z, axis=1, keepdims=True)
    e = z - m
    out[...] = e - jnp.log(jnp.sum(jnp.exp(e), axis=1, keepdims=True))


_GM = N // BM

_d1_call = pl.pallas_call(
    _d1_body,
    grid=(_GM,),
    in_specs=[
        pl.BlockSpec((2, BM, IN_CH), lambda i: (0, i, 0)),
        pl.BlockSpec((BM, 1), lambda i: (i, 0)),
        pl.BlockSpec((BM, IN_CH), lambda i: (i, 0)),
        pl.BlockSpec((IN_CH, HID_CH), lambda i: (0, 0)),
        pl.BlockSpec((1, HID_CH), lambda i: (0, 0)),
        pl.BlockSpec((IN_CH, HID_CH), lambda i: (0, 0)),
    ],
    out_specs=pl.BlockSpec((2, BM, IN_CH), lambda i: (0, i, 0)),
    out_shape=jax.ShapeDtypeStruct((2, N, IN_CH), jnp.float32),
)

_d2_call = pl.pallas_call(
    _d2_body,
    grid=(_GM,),
    in_specs=[
        pl.BlockSpec((2, BM, IN_CH), lambda i: (0, i, 0)),
        pl.BlockSpec((BM, 1), lambda i: (i, 0)),
        pl.BlockSpec((2, BM, IN_CH), lambda i: (0, i, 0)),
        pl.BlockSpec((HID_CH, HID_CH), lambda i: (0, 0)),
        pl.BlockSpec((1, HID_CH), lambda i: (0, 0)),
        pl.BlockSpec((HID_CH, HID_CH), lambda i: (0, 0)),
        pl.BlockSpec((HID_CH, OUT_CH), lambda i: (0, 0)),
        pl.BlockSpec((1, OUT_CH), lambda i: (0, 0)),
    ],
    out_specs=pl.BlockSpec((BM, OUT_CH), lambda i: (i, 0)),
    out_shape=jax.ShapeDtypeStruct((N, OUT_CH), jnp.float32),
)


def kernel(x, edge_index, Wl1, bl1, Wr1, Wl2, bl2, Wr2, Wfc, bfc):
    src = edge_index[0].astype(jnp.int32)
    dst = edge_index[1].astype(jnp.int32)
    src3 = src.reshape(NW, C1, K1)
    dst3 = dst.reshape(NW, C1, K1)
    srcB = src.reshape(NS, C2, K2)
    dstB = dst.reshape(NS, C2, K2)
    zf = jnp.zeros((NP, IN_CH), jnp.float32)
    z1 = jnp.zeros((NP,), jnp.float32)

    sc_layer1, sc_layer2 = _sc_kernels()
    aggp, cntp = sc_layer1(x, src3, dst3, zf, z1)
    rcnt = (1.0 / jnp.clip(cntp[0] + cntp[1], 1.0, None))[:, None]
    h1s = _d1_call(aggp, rcnt, x, Wl1, bl1.reshape(1, -1), Wr1)
    agg2 = sc_layer2(h1s, srcB, dstB, zf)
    out = _d2_call(agg2, rcnt, h1s, Wl2, bl2.reshape(1, -1), Wr2,
                   Wfc, bfc.reshape(1, -1))
    return out


# speedup vs baseline: 9.0310x; 1.1192x over previous
import functools

import jax
import jax.numpy as jnp
from jax import lax
from jax.experimental import pallas as pl
from jax.experimental.pallas import tpu as pltpu
from jax.experimental.pallas import tpu_sc as plsc

N = 10000
E = 320000
IN_CH = 128
HID_CH = 256
OUT_CH = 64

NC = 2
NS = 16
NW = NC * NS

K1 = 125
C1 = E // NW // K1
K2 = 125
C2 = E // NS // K2
NP = 10240
ROWS_T = NP // NS

@functools.lru_cache(maxsize=None)
def _sc_kernels():
    mesh = plsc.VectorSubcoreMesh(core_axis_name="c", subcore_axis_name="s",
                                  num_cores=NC, num_subcores=NS)

    @functools.partial(
        pl.kernel,
        out_type=(
            jax.ShapeDtypeStruct((NC, NP, IN_CH), jnp.float32),
            jax.ShapeDtypeStruct((NC, NP), jnp.float32),
        ),
        mesh=mesh,
        scratch_types=[
            pltpu.VMEM((2, K1), jnp.int32),
            pltpu.VMEM((2, K1), jnp.int32),
            pltpu.VMEM((K1, IN_CH), jnp.float32),
            pltpu.VMEM((K1, IN_CH), jnp.float32),
            pltpu.VMEM((128,), jnp.float32),
            pltpu.VMEM_SHARED((NP, IN_CH), jnp.float32),
            pltpu.VMEM_SHARED((NP,), jnp.float32),
            pltpu.SemaphoreType.DMA,
            pltpu.SemaphoreType.DMA,
            pltpu.SemaphoreType.DMA,
            pltpu.SemaphoreType.DMA,
        ],
    )
    def sc_layer1(x_hbm, sd3, zf, z1, aggp, cntp,
                  sd0, sd1, rv0, rv1, onesv, acc, cacc,
                  gs0, gs1, ss0, ss1):
        c = lax.axis_index("c")
        s = lax.axis_index("s")
        w = c * NS + s
        r0 = s * ROWS_T
        sd = (sd0, sd1)
        rv = (rv0, rv1)
        gs = (gs0, gs1)
        ss = (ss0, ss1)
        pltpu.sync_copy(zf.at[pl.ds(r0, ROWS_T)], acc.at[pl.ds(r0, ROWS_T)])
        pltpu.sync_copy(z1.at[pl.ds(r0, ROWS_T)], cacc.at[pl.ds(r0, ROWS_T)])
        ones16 = jnp.ones((16,), jnp.float32)
        for i in range(8):
            onesv[pl.ds(i * 16, 16)] = ones16
        plsc.subcore_barrier()

        def issue_gather(j, b):
            pltpu.sync_copy(sd3.at[w].at[j], sd[b])
            pltpu.async_copy(x_hbm.at[sd[b].at[0]], rv[b], gs[b])

        def issue_scatter(b):
            pltpu.async_copy(rv[b], acc.at[sd[b].at[1]], ss[b], add=True)
            pltpu.async_copy(onesv.at[pl.ds(0, K1)], cacc.at[sd[b].at[1]],
                             ss[b], add=True)

        def wait_gather(b):
            pltpu.make_async_copy(x_hbm.at[sd[b].at[0]], rv[b], gs[b]).wait()

        def wait_scatter(b):
            pltpu.make_async_copy(rv[b], acc.at[sd[b].at[1]], ss[b]).wait()
            pltpu.make_async_copy(onesv.at[pl.ds(0, K1)],
                                  cacc.at[sd[b].at[1]], ss[b]).wait()

        issue_gather(0, 0)
        issue_gather(1, 1)

        def body(k, carry):
            for b in range(2):
                wait_gather(b)
                issue_scatter(b)
            for b in range(2):
                wait_scatter(b)
                issue_gather(2 * k + 2 + b, b)
            return carry

        lax.fori_loop(0, C1 // 2 - 1, body, 0)
        for b in range(2):
            wait_gather(b)
            issue_scatter(b)
        for b in range(2):
            wait_scatter(b)
        plsc.subcore_barrier()
        pltpu.sync_copy(acc.at[pl.ds(r0, ROWS_T)],
                        aggp.at[c].at[pl.ds(r0, ROWS_T)])
        pltpu.sync_copy(cacc.at[pl.ds(r0, ROWS_T)],
                        cntp.at[c].at[pl.ds(r0, ROWS_T)])

    @functools.partial(
        pl.kernel,
        out_type=jax.ShapeDtypeStruct((NC, NP, IN_CH), jnp.float32),
        mesh=mesh,
        scratch_types=[
            pltpu.VMEM((2, K2), jnp.int32),
            pltpu.VMEM((2, K2), jnp.int32),
            pltpu.VMEM((K2, IN_CH), jnp.float32),
            pltpu.VMEM((K2, IN_CH), jnp.float32),
            pltpu.VMEM_SHARED((NP, IN_CH), jnp.float32),
            pltpu.SemaphoreType.DMA,
            pltpu.SemaphoreType.DMA,
            pltpu.SemaphoreType.DMA,
            pltpu.SemaphoreType.DMA,
        ],
    )
    def sc_layer2(h1s_hbm, sdB, zf, agg2,
                  sd0, sd1, rv0, rv1, acc, gs0, gs1, ss0, ss1):
        c = lax.axis_index("c")
        s = lax.axis_index("s")
        r0 = s * ROWS_T
        sd = (sd0, sd1)
        rv = (rv0, rv1)
        gs = (gs0, gs1)
        ss = (ss0, ss1)
        pltpu.sync_copy(zf.at[pl.ds(r0, ROWS_T)], acc.at[pl.ds(r0, ROWS_T)])
        plsc.subcore_barrier()

        def issue_gather(j, b):
            pltpu.sync_copy(sdB.at[s].at[j], sd[b])
            pltpu.async_copy(h1s_hbm.at[c].at[sd[b].at[0]], rv[b], gs[b])

        issue_gather(0, 0)
        issue_gather(1, 1)

        def body(k, carry):
            for b in range(2):
                pltpu.make_async_copy(h1s_hbm.at[c].at[sd[b].at[0]], rv[b],
                                      gs[b]).wait()
                pltpu.async_copy(rv[b], acc.at[sd[b].at[1]], ss[b], add=True)
            for b in range(2):
                pltpu.make_async_copy(rv[b], acc.at[sd[b].at[1]], ss[b]).wait()
                issue_gather(2 * k + 2 + b, b)
            return carry

        lax.fori_loop(0, C2 // 2 - 1, body, 0)
        for b in range(2):
            pltpu.make_async_copy(h1s_hbm.at[c].at[sd[b].at[0]], rv[b],
                                  gs[b]).wait()
            pltpu.async_copy(rv[b], acc.at[sd[b].at[1]], ss[b], add=True)
        for b in range(2):
            pltpu.make_async_copy(rv[b], acc.at[sd[b].at[1]], ss[b]).wait()
        plsc.subcore_barrier()
        pltpu.sync_copy(acc.at[pl.ds(r0, ROWS_T)],
                        agg2.at[c].at[pl.ds(r0, ROWS_T)])

    return sc_layer1, sc_layer2


BM = 2000


def _d1_body(aggp, rcnt, x, wl, bl, wr, h1s):
    agg = aggp[0] + aggp[1]
    mw = lax.dot(agg, wl[...], precision=lax.Precision.HIGHEST) * rcnt[...]
    h = mw + bl[...] + lax.dot(x[...], wr[...], precision=lax.Precision.HIGHEST)
    h = jnp.maximum(h, 0.0)
    h1s[0] = h[:, :IN_CH]
    h1s[1] = h[:, IN_CH:]


def _d2_body(agg2, rcnt, h1s, wl, bl, wr, wfc, bfc, out):
    agg = jnp.concatenate([agg2[0], agg2[1]], axis=1)
    h1 = jnp.concatenate([h1s[0], h1s[1]], axis=1)
    mw = lax.dot(agg, wl[...], precision=lax.Precision.HIGHEST) * rcnt[...]
    h2 = mw + bl[...] + lax.dot(h1, wr[...], precision=lax.Precision.HIGHEST)
    h2 = jnp.maximum(h2, 0.0)
    z = lax.dot(h2, wfc[...], precision=lax.Precision.HIGHEST) + bfc[...]
    m = jnp.max(z, axis=1, keepdims=True)
    e = z - m
    out[...] = e - jnp.log(jnp.sum(jnp.exp(e), axis=1, keepdims=True))


_GM = N // BM

_d1_call = pl.pallas_call(
    _d1_body,
    grid=(_GM,),
    in_specs=[
        pl.BlockSpec((2, BM, IN_CH), lambda i: (0, i, 0)),
        pl.BlockSpec((BM, 1), lambda i: (i, 0)),
        pl.BlockSpec((BM, IN_CH), lambda i: (i, 0)),
        pl.BlockSpec((IN_CH, HID_CH), lambda i: (0, 0)),
        pl.BlockSpec((1, HID_CH), lambda i: (0, 0)),
        pl.BlockSpec((IN_CH, HID_CH), lambda i: (0, 0)),
    ],
    out_specs=pl.BlockSpec((2, BM, IN_CH), lambda i: (0, i, 0)),
    out_shape=jax.ShapeDtypeStruct((2, N, IN_CH), jnp.float32),
)

_d2_call = pl.pallas_call(
    _d2_body,
    grid=(_GM,),
    in_specs=[
        pl.BlockSpec((2, BM, IN_CH), lambda i: (0, i, 0)),
        pl.BlockSpec((BM, 1), lambda i: (i, 0)),
        pl.BlockSpec((2, BM, IN_CH), lambda i: (0, i, 0)),
        pl.BlockSpec((HID_CH, HID_CH), lambda i: (0, 0)),
        pl.BlockSpec((1, HID_CH), lambda i: (0, 0)),
        pl.BlockSpec((HID_CH, HID_CH), lambda i: (0, 0)),
        pl.BlockSpec((HID_CH, OUT_CH), lambda i: (0, 0)),
        pl.BlockSpec((1, OUT_CH), lambda i: (0, 0)),
    ],
    out_specs=pl.BlockSpec((BM, OUT_CH), lambda i: (i, 0)),
    out_shape=jax.ShapeDtypeStruct((N, OUT_CH), jnp.float32),
)


def kernel(x, edge_index, Wl1, bl1, Wr1, Wl2, bl2, Wr2, Wfc, bfc):
    src = edge_index[0].astype(jnp.int32)
    dst = edge_index[1].astype(jnp.int32)
    sd3 = jnp.stack([src.reshape(NW, C1, K1), dst.reshape(NW, C1, K1)],
                    axis=2)
    sdB = jnp.stack([src.reshape(NS, C2, K2), dst.reshape(NS, C2, K2)],
                    axis=2)
    zf = jnp.zeros((NP, IN_CH), jnp.float32)
    z1 = jnp.zeros((NP,), jnp.float32)

    sc_layer1, sc_layer2 = _sc_kernels()
    aggp, cntp = sc_layer1(x, sd3, zf, z1)
    rcnt = (1.0 / jnp.clip(cntp[0] + cntp[1], 1.0, None))[:, None]
    h1s = _d1_call(aggp, rcnt, x, Wl1, bl1.reshape(1, -1), Wr1)
    agg2 = sc_layer2(h1s, sdB, zf)
    out = _d2_call(agg2, rcnt, h1s, Wl2, bl2.reshape(1, -1), Wr2,
                   Wfc, bfc.reshape(1, -1))
    return out


# speedup vs baseline: 10.5977x; 1.1735x over previous
import functools

import jax
import jax.numpy as jnp
from jax import lax
from jax.experimental import pallas as pl
from jax.experimental.pallas import tpu as pltpu
from jax.experimental.pallas import tpu_sc as plsc

N = 10000
E = 320000
IN_CH = 128
HID_CH = 256
OUT_CH = 64

NC = 2
NS = 16
NW = NC * NS

K1 = 125
C1 = E // NW // K1
K2 = 125
C2 = E // NS // K2
NP = 10240
ROWS_T = NP // NS

def _edge_pipeline(C, ld_idx, ig, wg, isc, wsc):
    assert C % 4 == 0 and (C // 2) % 2 == 0
    ld_idx(0, 0, 0)
    ld_idx(1, 1, 0)
    ig(0, 0)
    ld_idx(2, 0, 1)
    ig(1, 0)
    wg(0, 0)
    isc(0, 0)
    ld_idx(3, 1, 1)

    steps = (
        (0, 1, 0, 1, 0, 0),
        (1, 1, 0, 0, 1, 0),
        (0, 0, 1, 1, 1, 1),
        (1, 0, 1, 0, 0, 1),
    )

    def body(k, carry):
        j0 = 4 * k + 2
        for i, (b, q, wsq, wgb, wgq, pfq) in enumerate(steps):
            wsc(b, wsq)
            ig(b, q)
            wg(wgb, wgq)
            isc(wgb, wgq)
            ld_idx(j0 + i + 2, b, pfq)
        return carry

    lax.fori_loop(0, (C - 4) // 4, body, 0)
    wsc(0, 0)
    ig(0, 1)
    wg(1, 0)
    isc(1, 0)
    wsc(1, 0)
    ig(1, 1)
    wg(0, 1)
    isc(0, 1)
    wg(1, 1)
    isc(1, 1)
    wsc(0, 1)
    wsc(1, 1)


@functools.lru_cache(maxsize=None)
def _sc_kernels():
    mesh = plsc.VectorSubcoreMesh(core_axis_name="c", subcore_axis_name="s",
                                  num_cores=NC, num_subcores=NS)

    @functools.partial(
        pl.kernel,
        out_type=(
            jax.ShapeDtypeStruct((NC, NP, IN_CH), jnp.float32),
            jax.ShapeDtypeStruct((NC, NP), jnp.float32),
        ),
        mesh=mesh,
        scratch_types=[
            pltpu.VMEM((2, K1), jnp.int32),
            pltpu.VMEM((2, K1), jnp.int32),
            pltpu.VMEM((2, K1), jnp.int32),
            pltpu.VMEM((2, K1), jnp.int32),
            pltpu.VMEM((K1, IN_CH), jnp.float32),
            pltpu.VMEM((K1, IN_CH), jnp.float32),
            pltpu.VMEM((128,), jnp.float32),
            pltpu.VMEM_SHARED((NP, IN_CH), jnp.float32),
            pltpu.VMEM_SHARED((NP,), jnp.float32),
            pltpu.SemaphoreType.DMA,
            pltpu.SemaphoreType.DMA,
            pltpu.SemaphoreType.DMA,
            pltpu.SemaphoreType.DMA,
        ],
    )
    def sc_layer1(x_hbm, sd3, zf, z1, aggp, cntp,
                  sd00, sd01, sd10, sd11, rv0, rv1, onesv, acc, cacc,
                  gs0, gs1, ss0, ss1):
        c = lax.axis_index("c")
        s = lax.axis_index("s")
        w = c * NS + s
        r0 = s * ROWS_T
        sd = ((sd00, sd01), (sd10, sd11))
        rv = (rv0, rv1)
        gs = (gs0, gs1)
        ss = (ss0, ss1)
        pltpu.sync_copy(zf, acc.at[pl.ds(r0, ROWS_T)])
        pltpu.sync_copy(z1, cacc.at[pl.ds(r0, ROWS_T)])
        ones16 = jnp.ones((16,), jnp.float32)
        for i in range(8):
            onesv[pl.ds(i * 16, 16)] = ones16
        plsc.subcore_barrier()

        def ld_idx(j, b, q):
            pltpu.sync_copy(sd3.at[w].at[j], sd[b][q])

        def ig(b, q):
            pltpu.async_copy(x_hbm.at[sd[b][q].at[0]], rv[b], gs[b])

        def wg(b, q):
            pltpu.make_async_copy(x_hbm.at[sd[b][q].at[0]], rv[b],
                                  gs[b]).wait()

        def isc(b, q):
            pltpu.async_copy(rv[b], acc.at[sd[b][q].at[1]], ss[b], add=True)
            pltpu.async_copy(onesv.at[pl.ds(0, K1)], cacc.at[sd[b][q].at[1]],
                             ss[b], add=True)

        def wsc(b, q):
            pltpu.make_async_copy(rv[b], acc.at[sd[b][q].at[1]],
                                  ss[b]).wait()
            pltpu.make_async_copy(onesv.at[pl.ds(0, K1)],
                                  cacc.at[sd[b][q].at[1]], ss[b]).wait()

        _edge_pipeline(C1, ld_idx, ig, wg, isc, wsc)
        plsc.subcore_barrier()
        pltpu.sync_copy(acc.at[pl.ds(r0, ROWS_T)],
                        aggp.at[c].at[pl.ds(r0, ROWS_T)])
        pltpu.sync_copy(cacc.at[pl.ds(r0, ROWS_T)],
                        cntp.at[c].at[pl.ds(r0, ROWS_T)])

    @functools.partial(
        pl.kernel,
        out_type=jax.ShapeDtypeStruct((NC, NP, IN_CH), jnp.float32),
        mesh=mesh,
        scratch_types=[
            pltpu.VMEM((2, K2), jnp.int32),
            pltpu.VMEM((2, K2), jnp.int32),
            pltpu.VMEM((2, K2), jnp.int32),
            pltpu.VMEM((2, K2), jnp.int32),
            pltpu.VMEM((K2, IN_CH), jnp.float32),
            pltpu.VMEM((K2, IN_CH), jnp.float32),
            pltpu.VMEM_SHARED((NP, IN_CH), jnp.float32),
            pltpu.SemaphoreType.DMA,
            pltpu.SemaphoreType.DMA,
            pltpu.SemaphoreType.DMA,
            pltpu.SemaphoreType.DMA,
        ],
    )
    def sc_layer2(h1s_hbm, sdB, zf, agg2,
                  sd00, sd01, sd10, sd11, rv0, rv1, acc, gs0, gs1, ss0, ss1):
        c = lax.axis_index("c")
        s = lax.axis_index("s")
        r0 = s * ROWS_T
        sd = ((sd00, sd01), (sd10, sd11))
        rv = (rv0, rv1)
        gs = (gs0, gs1)
        ss = (ss0, ss1)
        pltpu.sync_copy(zf, acc.at[pl.ds(r0, ROWS_T)])
        plsc.subcore_barrier()

        def ld_idx(j, b, q):
            pltpu.sync_copy(sdB.at[s].at[j], sd[b][q])

        def ig(b, q):
            pltpu.async_copy(h1s_hbm.at[c].at[sd[b][q].at[0]], rv[b], gs[b])

        def wg(b, q):
            pltpu.make_async_copy(h1s_hbm.at[c].at[sd[b][q].at[0]], rv[b],
                                  gs[b]).wait()

        def isc(b, q):
            pltpu.async_copy(rv[b], acc.at[sd[b][q].at[1]], ss[b], add=True)

        def wsc(b, q):
            pltpu.make_async_copy(rv[b], acc.at[sd[b][q].at[1]],
                                  ss[b]).wait()

        _edge_pipeline(C2, ld_idx, ig, wg, isc, wsc)
        plsc.subcore_barrier()
        pltpu.sync_copy(acc.at[pl.ds(r0, ROWS_T)],
                        agg2.at[c].at[pl.ds(r0, ROWS_T)])

    return sc_layer1, sc_layer2


BM = 2000


def _d1_body(aggp, rcnt, x, wl, bl, wr, h1s):
    agg = aggp[0] + aggp[1]
    mw = lax.dot(agg, wl[...], precision=lax.Precision.HIGHEST) * rcnt[...]
    h = mw + bl[...] + lax.dot(x[...], wr[...], precision=lax.Precision.HIGHEST)
    h = jnp.maximum(h, 0.0)
    h1s[0] = h[:, :IN_CH]
    h1s[1] = h[:, IN_CH:]


def _d2_body(agg2, rcnt, h1s, wl, bl, wr, wfc, bfc, out):
    agg = jnp.concatenate([agg2[0], agg2[1]], axis=1)
    h1 = jnp.concatenate([h1s[0], h1s[1]], axis=1)
    mw = lax.dot(agg, wl[...], precision=lax.Precision.HIGHEST) * rcnt[...]
    h2 = mw + bl[...] + lax.dot(h1, wr[...], precision=lax.Precision.HIGHEST)
    h2 = jnp.maximum(h2, 0.0)
    z = lax.dot(h2, wfc[...], precision=lax.Precision.HIGHEST) + bfc[...]
    m = jnp.max(z, axis=1, keepdims=True)
    e = z - m
    out[...] = e - jnp.log(jnp.sum(jnp.exp(e), axis=1, keepdims=True))


_GM = N // BM

_d1_call = pl.pallas_call(
    _d1_body,
    grid=(_GM,),
    in_specs=[
        pl.BlockSpec((2, BM, IN_CH), lambda i: (0, i, 0)),
        pl.BlockSpec((BM, 1), lambda i: (i, 0)),
        pl.BlockSpec((BM, IN_CH), lambda i: (i, 0)),
        pl.BlockSpec((IN_CH, HID_CH), lambda i: (0, 0)),
        pl.BlockSpec((1, HID_CH), lambda i: (0, 0)),
        pl.BlockSpec((IN_CH, HID_CH), lambda i: (0, 0)),
    ],
    out_specs=pl.BlockSpec((2, BM, IN_CH), lambda i: (0, i, 0)),
    out_shape=jax.ShapeDtypeStruct((2, N, IN_CH), jnp.float32),
)

_d2_call = pl.pallas_call(
    _d2_body,
    grid=(_GM,),
    in_specs=[
        pl.BlockSpec((2, BM, IN_CH), lambda i: (0, i, 0)),
        pl.BlockSpec((BM, 1), lambda i: (i, 0)),
        pl.BlockSpec((2, BM, IN_CH), lambda i: (0, i, 0)),
        pl.BlockSpec((HID_CH, HID_CH), lambda i: (0, 0)),
        pl.BlockSpec((1, HID_CH), lambda i: (0, 0)),
        pl.BlockSpec((HID_CH, HID_CH), lambda i: (0, 0)),
        pl.BlockSpec((HID_CH, OUT_CH), lambda i: (0, 0)),
        pl.BlockSpec((1, OUT_CH), lambda i: (0, 0)),
    ],
    out_specs=pl.BlockSpec((BM, OUT_CH), lambda i: (i, 0)),
    out_shape=jax.ShapeDtypeStruct((N, OUT_CH), jnp.float32),
)


def kernel(x, edge_index, Wl1, bl1, Wr1, Wl2, bl2, Wr2, Wfc, bfc):
    src = edge_index[0].astype(jnp.int32)
    dst = edge_index[1].astype(jnp.int32)
    sd = jnp.stack([src.reshape(-1, K1), dst.reshape(-1, K1)], axis=1)
    sd3 = sd.reshape(NW, C1, 2, K1)
    sdB = sd.reshape(NS, C2, 2, K2)
    zf = jnp.zeros((ROWS_T, IN_CH), jnp.float32)
    z1 = jnp.zeros((ROWS_T,), jnp.float32)

    sc_layer1, sc_layer2 = _sc_kernels()
    aggp, cntp = sc_layer1(x, sd3, zf, z1)
    rcnt = (1.0 / jnp.clip(cntp[0] + cntp[1], 1.0, None))[:, None]
    h1s = _d1_call(aggp, rcnt, x, Wl1, bl1.reshape(1, -1), Wr1)
    agg2 = sc_layer2(h1s, sdB, zf)
    out = _d2_call(agg2, rcnt, h1s, Wl2, bl2.reshape(1, -1), Wr2,
                   Wfc, bfc.reshape(1, -1))
    return out


# speedup vs baseline: 10.7761x; 1.0168x over previous
import functools

import jax
import jax.numpy as jnp
from jax import lax
from jax.experimental import pallas as pl
from jax.experimental.pallas import tpu as pltpu
from jax.experimental.pallas import tpu_sc as plsc

N = 10000
E = 320000
IN_CH = 128
HID_CH = 256
OUT_CH = 64

NC = 2
NS = 16
NW = NC * NS

K1 = 125
C1 = E // NW // K1
K2 = 125
C2 = E // NS // K2
NP = 10240
ROWS_T = NP // NS

def _edge_pipeline(C, ld_idx, ig, wg, isc, wsc):
    assert C % 4 == 0 and (C // 2) % 2 == 0
    ld_idx(0, 0, 0)
    ld_idx(1, 1, 0)
    ig(0, 0)
    ld_idx(2, 0, 1)
    ig(1, 0)
    wg(0, 0)
    isc(0, 0)
    ld_idx(3, 1, 1)

    steps = (
        (0, 1, 0, 1, 0, 0),
        (1, 1, 0, 0, 1, 0),
        (0, 0, 1, 1, 1, 1),
        (1, 0, 1, 0, 0, 1),
    )

    def body(k, carry):
        j0 = 4 * k + 2
        for i, (b, q, wsq, wgb, wgq, pfq) in enumerate(steps):
            wsc(b, wsq)
            ig(b, q)
            wg(wgb, wgq)
            isc(wgb, wgq)
            ld_idx(j0 + i + 2, b, pfq)
        return carry

    lax.fori_loop(0, (C - 4) // 4, body, 0)
    wsc(0, 0)
    ig(0, 1)
    wg(1, 0)
    isc(1, 0)
    wsc(1, 0)
    ig(1, 1)
    wg(0, 1)
    isc(0, 1)
    wg(1, 1)
    isc(1, 1)
    wsc(0, 1)
    wsc(1, 1)


@functools.lru_cache(maxsize=None)
def _sc_kernels():
    mesh = plsc.VectorSubcoreMesh(core_axis_name="c", subcore_axis_name="s",
                                  num_cores=NC, num_subcores=NS)

    @functools.partial(
        pl.kernel,
        out_type=(
            jax.ShapeDtypeStruct((NC, NP, IN_CH), jnp.float32),
            jax.ShapeDtypeStruct((NC, NP), jnp.float32),
        ),
        mesh=mesh,
        scratch_types=[
            pltpu.VMEM((2, K1), jnp.int32),
            pltpu.VMEM((2, K1), jnp.int32),
            pltpu.VMEM((2, K1), jnp.int32),
            pltpu.VMEM((2, K1), jnp.int32),
            pltpu.VMEM((K1, IN_CH), jnp.float32),
            pltpu.VMEM((K1, IN_CH), jnp.float32),
            pltpu.VMEM((128,), jnp.float32),
            pltpu.VMEM_SHARED((NP, IN_CH), jnp.float32),
            pltpu.VMEM_SHARED((NP,), jnp.float32),
            pltpu.SemaphoreType.DMA,
            pltpu.SemaphoreType.DMA,
            pltpu.SemaphoreType.DMA,
            pltpu.SemaphoreType.DMA,
        ],
    )
    def sc_layer1(x_hbm, sd3, zf, z1, aggp, cntp,
                  sd00, sd01, sd10, sd11, rv0, rv1, onesv, acc, cacc,
                  gs0, gs1, ss0, ss1):
        c = lax.axis_index("c")
        s = lax.axis_index("s")
        w = c * NS + s
        r0 = s * ROWS_T
        sd = ((sd00, sd01), (sd10, sd11))
        rv = (rv0, rv1)
        gs = (gs0, gs1)
        ss = (ss0, ss1)
        pltpu.sync_copy(zf, acc.at[pl.ds(r0, ROWS_T)])
        pltpu.sync_copy(z1, cacc.at[pl.ds(r0, ROWS_T)])
        ones16 = jnp.ones((16,), jnp.float32)
        for i in range(8):
            onesv[pl.ds(i * 16, 16)] = ones16
        plsc.subcore_barrier()

        def ld_idx(j, b, q):
            pltpu.sync_copy(sd3.at[w].at[j], sd[b][q])

        def ig(b, q):
            pltpu.async_copy(x_hbm.at[sd[b][q].at[0]], rv[b], gs[b])

        def wg(b, q):
            pltpu.make_async_copy(x_hbm.at[sd[b][q].at[0]], rv[b],
                                  gs[b]).wait()

        def isc(b, q):
            pltpu.async_copy(rv[b], acc.at[sd[b][q].at[1]], ss[b], add=True)
            pltpu.async_copy(onesv.at[pl.ds(0, K1)], cacc.at[sd[b][q].at[1]],
                             ss[b], add=True)

        def wsc(b, q):
            pltpu.make_async_copy(rv[b], acc.at[sd[b][q].at[1]],
                                  ss[b]).wait()
            pltpu.make_async_copy(onesv.at[pl.ds(0, K1)],
                                  cacc.at[sd[b][q].at[1]], ss[b]).wait()

        _edge_pipeline(C1, ld_idx, ig, wg, isc, wsc)
        plsc.subcore_barrier()
        pltpu.sync_copy(acc.at[pl.ds(r0, ROWS_T)],
                        aggp.at[c].at[pl.ds(r0, ROWS_T)])
        pltpu.sync_copy(cacc.at[pl.ds(r0, ROWS_T)],
                        cntp.at[c].at[pl.ds(r0, ROWS_T)])

    @functools.partial(
        pl.kernel,
        out_type=jax.ShapeDtypeStruct((NC, NP, IN_CH), jnp.float32),
        mesh=mesh,
        scratch_types=[
            pltpu.VMEM((2, K2), jnp.int32),
            pltpu.VMEM((2, K2), jnp.int32),
            pltpu.VMEM((2, K2), jnp.int32),
            pltpu.VMEM((2, K2), jnp.int32),
            pltpu.VMEM((K2, IN_CH), jnp.float32),
            pltpu.VMEM((K2, IN_CH), jnp.float32),
            pltpu.VMEM_SHARED((NP, IN_CH), jnp.float32),
            pltpu.SemaphoreType.DMA,
            pltpu.SemaphoreType.DMA,
            pltpu.SemaphoreType.DMA,
            pltpu.SemaphoreType.DMA,
        ],
    )
    def sc_layer2(h1s_hbm, sdB, zf, agg2,
                  sd00, sd01, sd10, sd11, rv0, rv1, acc, gs0, gs1, ss0, ss1):
        c = lax.axis_index("c")
        s = lax.axis_index("s")
        r0 = s * ROWS_T
        sd = ((sd00, sd01), (sd10, sd11))
        rv = (rv0, rv1)
        gs = (gs0, gs1)
        ss = (ss0, ss1)
        pltpu.sync_copy(zf, acc.at[pl.ds(r0, ROWS_T)])
        plsc.subcore_barrier()

        def ld_idx(j, b, q):
            pltpu.sync_copy(sdB.at[s].at[j], sd[b][q])

        def ig(b, q):
            pltpu.async_copy(h1s_hbm.at[c].at[sd[b][q].at[0]], rv[b], gs[b])

        def wg(b, q):
            pltpu.make_async_copy(h1s_hbm.at[c].at[sd[b][q].at[0]], rv[b],
                                  gs[b]).wait()

        def isc(b, q):
            pltpu.async_copy(rv[b], acc.at[sd[b][q].at[1]], ss[b], add=True)

        def wsc(b, q):
            pltpu.make_async_copy(rv[b], acc.at[sd[b][q].at[1]],
                                  ss[b]).wait()

        _edge_pipeline(C2, ld_idx, ig, wg, isc, wsc)
        plsc.subcore_barrier()
        pltpu.sync_copy(acc.at[pl.ds(r0, ROWS_T)],
                        agg2.at[c].at[pl.ds(r0, ROWS_T)])

    return sc_layer1, sc_layer2


BM = 2000
_GM = N // BM
_PREC = lax.Precision.HIGHEST


def _root_body(x, wr, bl, xr):
    xr[...] = lax.dot(x[...], wr[...], precision=_PREC) + bl[...]


def _root2_body(h1s, wr, bl, hr):
    h1 = jnp.concatenate([h1s[0], h1s[1]], axis=1)
    hr[...] = lax.dot(h1, wr[...], precision=_PREC) + bl[...]


def _d1_body(aggp, rcnt, xr, wl, h1s):
    agg = aggp[0] + aggp[1]
    mw = lax.dot(agg, wl[...], precision=_PREC) * rcnt[...]
    h = jnp.maximum(mw + xr[...], 0.0)
    h1s[0] = h[:, :IN_CH]
    h1s[1] = h[:, IN_CH:]


def _d2_body(agg2, rcnt, hr, wl, wfc, bfc, out):
    agg = jnp.concatenate([agg2[0], agg2[1]], axis=1)
    mw = lax.dot(agg, wl[...], precision=_PREC) * rcnt[...]
    h2 = jnp.maximum(mw + hr[...], 0.0)
    z = lax.dot(h2, wfc[...], precision=_PREC) + bfc[...]
    m = jnp.max(z, axis=1, keepdims=True)
    e = z - m
    out[...] = e - jnp.log(jnp.sum(jnp.exp(e), axis=1, keepdims=True))


_root_call = pl.pallas_call(
    _root_body,
    grid=(_GM,),
    in_specs=[
        pl.BlockSpec((BM, IN_CH), lambda i: (i, 0)),
        pl.BlockSpec((IN_CH, HID_CH), lambda i: (0, 0)),
        pl.BlockSpec((1, HID_CH), lambda i: (0, 0)),
    ],
    out_specs=pl.BlockSpec((BM, HID_CH), lambda i: (i, 0)),
    out_shape=jax.ShapeDtypeStruct((N, HID_CH), jnp.float32),
)

_root2_call = pl.pallas_call(
    _root2_body,
    grid=(_GM,),
    in_specs=[
        pl.BlockSpec((2, BM, IN_CH), lambda i: (0, i, 0)),
        pl.BlockSpec((HID_CH, HID_CH), lambda i: (0, 0)),
        pl.BlockSpec((1, HID_CH), lambda i: (0, 0)),
    ],
    out_specs=pl.BlockSpec((BM, HID_CH), lambda i: (i, 0)),
    out_shape=jax.ShapeDtypeStruct((N, HID_CH), jnp.float32),
)

_d1_call = pl.pallas_call(
    _d1_body,
    grid=(_GM,),
    in_specs=[
        pl.BlockSpec((2, BM, IN_CH), lambda i: (0, i, 0)),
        pl.BlockSpec((BM, 1), lambda i: (i, 0)),
        pl.BlockSpec((BM, HID_CH), lambda i: (i, 0)),
        pl.BlockSpec((IN_CH, HID_CH), lambda i: (0, 0)),
    ],
    out_specs=pl.BlockSpec((2, BM, IN_CH), lambda i: (0, i, 0)),
    out_shape=jax.ShapeDtypeStruct((2, N, IN_CH), jnp.float32),
)

_d2_call = pl.pallas_call(
    _d2_body,
    grid=(_GM,),
    in_specs=[
        pl.BlockSpec((2, BM, IN_CH), lambda i: (0, i, 0)),
        pl.BlockSpec((BM, 1), lambda i: (i, 0)),
        pl.BlockSpec((BM, HID_CH), lambda i: (i, 0)),
        pl.BlockSpec((HID_CH, HID_CH), lambda i: (0, 0)),
        pl.BlockSpec((HID_CH, OUT_CH), lambda i: (0, 0)),
        pl.BlockSpec((1, OUT_CH), lambda i: (0, 0)),
    ],
    out_specs=pl.BlockSpec((BM, OUT_CH), lambda i: (i, 0)),
    out_shape=jax.ShapeDtypeStruct((N, OUT_CH), jnp.float32),
)


def kernel(x, edge_index, Wl1, bl1, Wr1, Wl2, bl2, Wr2, Wfc, bfc):
    src = edge_index[0].astype(jnp.int32)
    dst = edge_index[1].astype(jnp.int32)
    sd = jnp.stack([src.reshape(-1, K1), dst.reshape(-1, K1)], axis=1)
    sd3 = sd.reshape(NW, C1, 2, K1)
    sdB = sd.reshape(NS, C2, 2, K2)
    zf = jnp.zeros((ROWS_T, IN_CH), jnp.float32)
    z1 = jnp.zeros((ROWS_T,), jnp.float32)

    sc_layer1, sc_layer2 = _sc_kernels()
    xr = _root_call(x, Wr1, bl1.reshape(1, -1))
    aggp, cntp = sc_layer1(x, sd3, zf, z1)
    rcnt = (1.0 / jnp.clip(cntp[0] + cntp[1], 1.0, None))[:, None]
    h1s = _d1_call(aggp, rcnt, xr, Wl1)
    hr = _root2_call(h1s, Wr2, bl2.reshape(1, -1))
    agg2 = sc_layer2(h1s, sdB, zf)
    out = _d2_call(agg2, rcnt, hr, Wl2, Wfc, bfc.reshape(1, -1))
    return out
